# Initial kernel scaffold; baseline (speedup 1.0000x reference)
import jax
import jax.numpy as jnp
from jax.experimental import pallas as pl


def kernel(h, x, edge_index, edge_weight, W_pre, b_pre, W_sage_l, b_sage, W_sage_r, W_arma_init, W_arma_root, b_arma):
    raise NotImplementedError("write your pallas kernel here")


# speedup vs baseline: 14.7434x; 14.7434x over previous
import functools

import jax
import jax.numpy as jnp
from jax import lax
from jax.experimental import pallas as pl
from jax.experimental.pallas import tpu as pltpu
from jax.experimental.pallas import tpu_sc as plsc

N = 10000
E = 320000
D = 128
H = 64
NP = 10112
NC = 2
NS = 16
CH = 64
NCHUNK = 5120
EP = NCHUNK * CH

def _pre_body(x_ref, w_ref, b_ref, o_ref):
    o_ref[0] = jnp.dot(x_ref[...], w_ref[0], preferred_element_type=jnp.float32) + b_ref[0]


def _dense_pre(x_pad, w2, b2):
    br = 1264
    grid = (2, NP // br)
    return pl.pallas_call(
        _pre_body,
        grid=grid,
        in_specs=[
            pl.BlockSpec((br, D), lambda c, i: (i, 0)),
            pl.BlockSpec((1, D, H), lambda c, i: (c, 0, 0)),
            pl.BlockSpec((1, 1, H), lambda c, i: (c, 0, 0)),
        ],
        out_specs=pl.BlockSpec((1, br, H), lambda c, i: (c, i, 0)),
        out_shape=jax.ShapeDtypeStruct((2, NP, H), jnp.float32),
    )(x_pad, w2, b2)


def _stats_body(dst_ref, ew_ref, ones_ref, z1_ref, deg_out, cnt_out,
                dstb0, dstb1, ewb0, ewb1, onev, deg_sh, cnt_sh, isem0, isem1):
    c = lax.axis_index("c")
    s = lax.axis_index("s")
    w = c * NS + s
    dstb = (dstb0, dstb1)
    ewb = (ewb0, ewb1)
    isem = (isem0, isem1)
    pltpu.sync_copy(ones_ref, onev)

    @pl.when(s == 0)
    def _():
        pltpu.sync_copy(z1_ref, deg_sh)
        pltpu.sync_copy(z1_ref, cnt_sh)

    plsc.subcore_barrier()

    cpw = NCHUNK // (NC * NS)

    def issue_idx(k, p):
        base = (w * cpw + k) * CH
        pltpu.async_copy(dst_ref.at[pl.ds(base, CH)], dstb[p], isem[p])
        pltpu.async_copy(ew_ref.at[pl.ds(base, CH)], ewb[p], isem[p])

    def wait_idx(k, p):
        base = (w * cpw + k) * CH
        pltpu.make_async_copy(dst_ref.at[pl.ds(base, CH)], dstb[p], isem[p]).wait()
        pltpu.make_async_copy(ew_ref.at[pl.ds(base, CH)], ewb[p], isem[p]).wait()

    issue_idx(0, 0)
    issue_idx(1, 1)

    def pair(m, _):
        for p in (0, 1):
            k = 2 * m + p
            wait_idx(k, p)
            pltpu.sync_copy(ewb[p], deg_sh.at[dstb[p]], add=True)
            pltpu.sync_copy(onev, cnt_sh.at[dstb[p]], add=True)

            @pl.when(k + 2 < cpw)
            def _():
                issue_idx(k + 2, p)
        return 0

    lax.fori_loop(0, cpw // 2, pair, 0)
    plsc.subcore_barrier()

    rpt = NP // NS
    pltpu.sync_copy(deg_sh.at[pl.ds(s * rpt, rpt)], deg_out.at[c, pl.ds(s * rpt, rpt)])
    pltpu.sync_copy(cnt_sh.at[pl.ds(s * rpt, rpt)], cnt_out.at[c, pl.ds(s * rpt, rpt)])


def _edge_stats(dst1, ew1, ones, z1):
    mesh = plsc.VectorSubcoreMesh(core_axis_name="c", subcore_axis_name="s",
                                  num_cores=NC, num_subcores=NS)
    f = pl.kernel(
        _stats_body,
        out_type=[jax.ShapeDtypeStruct((NC, NP), jnp.float32),
                  jax.ShapeDtypeStruct((NC, NP), jnp.float32)],
        mesh=mesh,
        compiler_params=pltpu.CompilerParams(
            needs_layout_passes=False, use_tc_tiling_on_sc=False),
        scratch_types=[
            pltpu.VMEM((CH,), jnp.int32),
            pltpu.VMEM((CH,), jnp.int32),
            pltpu.VMEM((CH,), jnp.float32),
            pltpu.VMEM((CH,), jnp.float32),
            pltpu.VMEM((CH,), jnp.float32),
            pltpu.VMEM_SHARED((NP,), jnp.float32),
            pltpu.VMEM_SHARED((NP,), jnp.float32),
            pltpu.SemaphoreType.DMA,
            pltpu.SemaphoreType.DMA,
        ],
    )
    return f(dst1, ew1, ones, z1)


def _prep_body(degp_ref, cntp_ref, dinv_ref, invc_ref):
    deg = degp_ref[0] + degp_ref[1]
    cnt = cntp_ref[0] + cntp_ref[1]
    dinv_ref[...] = jnp.where(deg > 0, lax.rsqrt(jnp.maximum(deg, 1e-12)), 0.0)
    invc_ref[...] = 1.0 / jnp.maximum(cnt, 1.0)


def _norm_prep(deg_p, cnt_p):
    r = NP // 128
    return pl.pallas_call(
        _prep_body,
        out_shape=[jax.ShapeDtypeStruct((r, 128), jnp.float32),
                   jax.ShapeDtypeStruct((r, 128), jnp.float32)],
    )(deg_p.reshape(2, r, 128), cnt_p.reshape(2, r, 128))


def _agg_body(xh_ref, src_ref, dst_ref, ew_ref, dinv_ref,
              out_ref,
              srcb0, srcb1, srcb2, srcb3, dstb0, dstb1, dstb2, dstb3,
              ewb0, ewb1, ewb2, ewb3, srcg0, srcg1, srcg2, srcg3,
              scix0, scix1, wsv0, wsv1, wsv2, wsv3, wnv0, wnv1, wnv2, wnv3,
              rows0, rows1, rows2, rows3, tmpm0, tmpm1, tmpp0, tmpp1,
              dinvt, accm, accp,
              isem0, isem1, isem2, isem3, gsem0, gsem1, gsem2, gsem3,
              ssem0, ssem1):
    c = lax.axis_index("c")
    s = lax.axis_index("s")
    srcb = (srcb0, srcb1, srcb2, srcb3)
    dstb = (dstb0, dstb1, dstb2, dstb3)
    ewb = (ewb0, ewb1, ewb2, ewb3)
    srcg = (srcg0, srcg1, srcg2, srcg3)
    scix = (scix0, scix1)
    wsv = (wsv0, wsv1, wsv2, wsv3)
    wnv = (wnv0, wnv1, wnv2, wnv3)
    rows = (rows0, rows1, rows2, rows3)
    tmpm = (tmpm0, tmpm1)
    tmpp = (tmpp0, tmpp1)
    isem = (isem0, isem1, isem2, isem3)
    gsem = (gsem0, gsem1, gsem2, gsem3)
    ssem = (ssem0, ssem1)

    cpt = NCHUNK // NS
    coff = c * NP

    pltpu.sync_copy(dinv_ref, dinvt)

    zvec = jnp.zeros((16,), jnp.float32)

    def zrow(r, _):
        for f in range(H // 16):
            tmpm0.at[r][pl.ds(f * 16, 16)] = zvec
        return 0

    lax.fori_loop(0, CH, zrow, 0)
    rpt = NP // NS
    nfull = rpt // CH
    for r in range(nfull):
        pltpu.sync_copy(tmpm0, accm.at[pl.ds(s * rpt + r * CH, CH)])
        pltpu.sync_copy(tmpm0, accp.at[pl.ds(s * rpt + r * CH, CH)])
    rem = rpt - nfull * CH
    if rem:
        pltpu.sync_copy(tmpm0.at[pl.ds(0, rem)], accm.at[pl.ds(s * rpt + nfull * CH, rem)])
        pltpu.sync_copy(tmpm0.at[pl.ds(0, rem)], accp.at[pl.ds(s * rpt + nfull * CH, rem)])

    plsc.subcore_barrier()

    def issue_idx(k, p):
        base = (s * cpt + k) * CH
        pltpu.async_copy(src_ref.at[pl.ds(base, CH)], srcb[p], isem[p])
        pltpu.async_copy(dst_ref.at[pl.ds(base, CH)], dstb[p], isem[p])
        pltpu.async_copy(ew_ref.at[pl.ds(base, CH)], ewb[p], isem[p])

    def wait_idx(k, p):
        base = (s * cpt + k) * CH
        pltpu.make_async_copy(src_ref.at[pl.ds(base, CH)], srcb[p], isem[p]).wait()
        pltpu.make_async_copy(dst_ref.at[pl.ds(base, CH)], dstb[p], isem[p]).wait()
        pltpu.make_async_copy(ew_ref.at[pl.ds(base, CH)], ewb[p], isem[p]).wait()

    def stage_a(p):
        for g in range(CH // 16):
            sl = pl.ds(g * 16, 16)
            si = srcb[p][sl]
            di = dstb[p][sl]
            wv = ewb[p][sl]
            dvs = plsc.load_gather(dinvt, [si])
            dvd = plsc.load_gather(dinvt, [di])
            wsv[p][sl] = wv
            wnv[p][sl] = dvs * wv * dvd
            srcg[p][sl] = si + coff

    def snap_scix(j, p):
        for g in range(CH // 16):
            sl = pl.ds(g * 16, 16)
            scix[p][sl] = dstb[j][sl]

    def issue_gather(p):
        pltpu.async_copy(xh_ref.at[srcg[p]], rows[p], gsem[p])

    def wait_gather(p):
        pltpu.make_async_copy(xh_ref.at[srcg[p]], rows[p], gsem[p]).wait()

    def issue_scatter(p):
        pltpu.async_copy(tmpm[p], accm.at[scix[p]], ssem[p], add=True)
        pltpu.async_copy(tmpp[p], accp.at[scix[p]], ssem[p], add=True)

    def wait_scatter(p):
        pltpu.make_async_copy(tmpm[p], accm.at[scix[p]], ssem[p]).wait()
        pltpu.make_async_copy(tmpp[p], accp.at[scix[p]], ssem[p]).wait()

    def stage_c(j, p):
        @plsc.parallel_loop(0, CH, 1, unroll=4)
        def _(e):
            spl = jnp.full((16,), e, jnp.int32)
            wse = plsc.load_gather(wsv[j], [spl])
            wne = plsc.load_gather(wnv[j], [spl])
            for f in range(H // 16):
                r = rows[j].at[e][pl.ds(f * 16, 16)]
                tmpm[p].at[e][pl.ds(f * 16, 16)] = r * wse
                tmpp[p].at[e][pl.ds(f * 16, 16)] = r * wne

    for kk in (0, 1):
        base0 = (s * cpt + kk) * CH
        pltpu.sync_copy(src_ref.at[pl.ds(base0, CH)], srcb[kk])
        pltpu.sync_copy(dst_ref.at[pl.ds(base0, CH)], dstb[kk])
        pltpu.sync_copy(ew_ref.at[pl.ds(base0, CH)], ewb[kk])
        stage_a(kk)
        issue_gather(kk)
    issue_idx(2, 2)
    issue_idx(3, 3)

    def quad(m, _):
        for j in (0, 1, 2, 3):
            k = 4 * m + j
            p = j % 2
            q = (j + 2) % 4

            @pl.when(k + 2 < cpt)
            def _():
                wait_idx(k + 2, q)
                stage_a(q)
                issue_gather(q)

            wait_gather(j)

            @pl.when(k >= 2)
            def _():
                wait_scatter(p)

            snap_scix(j, p)
            stage_c(j, p)
            issue_scatter(p)

            @pl.when(k + 4 < cpt)
            def _():
                issue_idx(k + 4, j)
        return 0

    lax.fori_loop(0, cpt // 4, quad, 0)
    wait_scatter(0)
    wait_scatter(1)
    plsc.subcore_barrier()

    pltpu.sync_copy(accm.at[pl.ds(s * rpt, rpt)], out_ref.at[c, 0, pl.ds(s * rpt, rpt)])
    pltpu.sync_copy(accp.at[pl.ds(s * rpt, rpt)], out_ref.at[c, 1, pl.ds(s * rpt, rpt)])


def _edge_agg(xh_cat, src1, dst2d, ew1, dinv):
    mesh = plsc.VectorSubcoreMesh(core_axis_name="c", subcore_axis_name="s",
                                  num_cores=NC, num_subcores=NS)
    cpt = NCHUNK // NS
    f = pl.kernel(
        _agg_body,
        out_type=[jax.ShapeDtypeStruct((NC, 2, NP, H), jnp.float32)],
        mesh=mesh,
        compiler_params=pltpu.CompilerParams(
            needs_layout_passes=False, use_tc_tiling_on_sc=False),
        scratch_types=(
            [pltpu.VMEM((CH,), jnp.int32)] * 4
            + [pltpu.VMEM((CH,), jnp.int32)] * 4
            + [pltpu.VMEM((CH,), jnp.float32)] * 4
            + [pltpu.VMEM((CH,), jnp.int32)] * 4
            + [pltpu.VMEM((CH,), jnp.int32)] * 2
            + [pltpu.VMEM((CH,), jnp.float32)] * 4
            + [pltpu.VMEM((CH,), jnp.float32)] * 4
            + [pltpu.VMEM((CH, H), jnp.float32)] * 4
            + [pltpu.VMEM((CH, H), jnp.float32)] * 4
            + [pltpu.VMEM((NP,), jnp.float32)]
            + [pltpu.VMEM_SHARED((NP, H), jnp.float32)] * 2
            + [pltpu.SemaphoreType.DMA] * 10
        ),
    )
    return f(xh_cat, src1, dst2d, ew1, dinv)


def _post_body(agg_ref, xh_ref, invc_ref, wl_ref, wr_ref, wa_ref, wt_ref,
               bs_ref, ba_ref, o_ref):
    dot = functools.partial(jnp.dot, preferred_element_type=jnp.float32)
    ic = invc_ref[...]
    m0 = agg_ref[0, 0] * ic
    m1 = agg_ref[1, 0] * ic
    p0 = agg_ref[0, 1]
    p1 = agg_ref[1, 1]
    sage = (dot(m0, wl_ref[0]) + dot(m1, wl_ref[1])
            + dot(xh_ref[0], wr_ref[0]) + dot(xh_ref[1], wr_ref[1]) + bs_ref[...])
    o1 = jnp.where(sage >= 0, sage, 0.01 * sage)
    pre = (dot(p0, wa_ref[0]) + dot(p1, wa_ref[1])
           + dot(xh_ref[0], wt_ref[0]) + dot(xh_ref[1], wt_ref[1]) + ba_ref[...])
    o_ref[...] = jnp.maximum(o1 + jnp.maximum(pre, 0.0), 0.0)


def _dense_post(agg, xh2, invc, wl2, wr2, wa2, wt2, bs, ba):
    br = 1264
    grid = (NP // br,)
    wspec = pl.BlockSpec((2, H, D), lambda i: (0, 0, 0))
    bspec = pl.BlockSpec((1, D), lambda i: (0, 0))
    return pl.pallas_call(
        _post_body,
        grid=grid,
        in_specs=[pl.BlockSpec((2, 2, br, H), lambda i: (0, 0, i, 0)),
                  pl.BlockSpec((2, br, H), lambda i: (0, i, 0)),
                  pl.BlockSpec((br, 1), lambda i: (i, 0)),
                  wspec, wspec, wspec, wspec, bspec, bspec],
        out_specs=pl.BlockSpec((br, D), lambda i: (i, 0)),
        out_shape=jax.ShapeDtypeStruct((NP, D), jnp.float32),
    )(agg, xh2, invc, wl2, wr2, wa2, wt2, bs, ba)


def kernel(h, x, edge_index, edge_weight, W_pre, b_pre, W_sage_l, b_sage,
           W_sage_r, W_arma_init, W_arma_root, b_arma):
    pad = EP - E
    src1 = jnp.concatenate([edge_index[0], jnp.full((pad,), N, jnp.int32)])
    dst1 = jnp.concatenate([edge_index[1], jnp.full((pad,), N, jnp.int32)])
    ew1 = jnp.concatenate([edge_weight, jnp.zeros((pad,), jnp.float32)])

    x_pad = jnp.pad(x, ((0, NP - N), (0, 0)))
    w2 = jnp.stack([W_pre[:H].T, W_pre[H:].T])
    b2 = jnp.stack([b_pre[:H], b_pre[H:]])[:, None, :]
    xh2 = _dense_pre(x_pad, w2, b2)

    ones = jnp.ones((CH,), jnp.float32)
    z1 = jnp.zeros((NP,), jnp.float32)
    deg_p, cnt_p = _edge_stats(dst1, ew1, ones, z1)

    dinv2d, invc2d = _norm_prep(deg_p, cnt_p)
    dinv = dinv2d.reshape(NP)
    invc = invc2d.reshape(NP, 1)

    xh_cat = xh2.reshape(2 * NP, H)
    agg, = _edge_agg(xh_cat, src1, dst1, ew1, dinv)

    wl2 = jnp.stack([W_sage_l[:, :H].T, W_sage_l[:, H:].T])
    wr2 = jnp.stack([W_sage_r[:, :H].T, W_sage_r[:, H:].T])
    wa2 = jnp.stack([W_arma_init[:H], W_arma_init[H:]])
    wt2 = jnp.stack([W_arma_root[:H], W_arma_root[H:]])
    o3p = _dense_post(agg, xh2, invc, wl2, wr2, wa2, wt2,
                      b_sage[None, :], b_arma[None, :])
    return (x, o3p[:N])
